# baseline (speedup 1.0000x reference)
import functools

import jax
import jax.numpy as jnp
from jax import lax
from jax.experimental import pallas as pl
from jax.experimental.pallas import tpu as pltpu
from jax.experimental.pallas import tpu_sc as plsc

N = 10000
E = 320000
H = 128
C = 16
G = 64

NC = 2
NS = 16
NW = NC * NS
K = 128
CPW = 80
EP = NW * CPW * K
NPH = 2
PH_ROWS = 5120
TRASH = PH_ROWS
ACC_ROWS = 5248
RPT = PH_ROWS // NS
DEG_ROWS = 10240
DEG_RPT = DEG_ROWS // NS

_mesh = plsc.VectorSubcoreMesh(
    core_axis_name="c", subcore_axis_name="s", num_cores=NC, num_subcores=NS
)


@functools.partial(
    pl.kernel,
    out_type=jax.ShapeDtypeStruct((NC, NPH, PH_ROWS, H), jnp.float32),
    mesh=_mesh,
    scratch_types=[
        pltpu.VMEM((CPW, K), jnp.int32),
        pltpu.VMEM((CPW, K), jnp.int32),
        pltpu.VMEM((K, H), jnp.float32),
        pltpu.VMEM((K, H), jnp.float32),
        pltpu.VMEM((128, H), jnp.float32),
        pltpu.VMEM_SHARED((ACC_ROWS, H), jnp.float32),
        pltpu.SemaphoreType.DMA,
        pltpu.SemaphoreType.DMA,
    ],
)
def _sc_scatter(g_hbm, src_hbm, dst0_hbm, dst1_hbm, out_hbm,
                src_v, dst_v, rows0, rows1, zb_v, acc, sem0, sem1):
    c = lax.axis_index("c")
    s = lax.axis_index("s")
    w = s * NC + c

    def fill_zero(r, _):
        for gcol in range(H // 16):
            zb_v[r, pl.ds(gcol * 16, 16)] = jnp.zeros((16,), jnp.float32)
        return 0

    lax.fori_loop(0, 128, fill_zero, 0)

    pltpu.sync_copy(src_hbm.at[w], src_v)

    for ph, dst_hbm in enumerate((dst0_hbm, dst1_hbm)):
        pltpu.sync_copy(dst_hbm.at[w], dst_v)
        pltpu.sync_copy(zb_v, acc.at[pl.ds(s * RPT, 128)])
        pltpu.sync_copy(zb_v, acc.at[pl.ds(s * RPT + 128, 128)])
        pltpu.sync_copy(zb_v.at[pl.ds(0, 64)], acc.at[pl.ds(s * RPT + 256, 64)])
        plsc.subcore_barrier()

        pltpu.async_copy(g_hbm.at[src_v.at[0]], rows0, sem0)

        def body(j, _):
            i0 = 2 * j
            pltpu.make_async_copy(g_hbm.at[src_v.at[i0]], rows0, sem0).wait()
            pltpu.async_copy(g_hbm.at[src_v.at[i0 + 1]], rows1, sem1)
            pltpu.sync_copy(rows0, acc.at[dst_v.at[i0]], add=True)
            pltpu.make_async_copy(g_hbm.at[src_v.at[i0 + 1]], rows1, sem1).wait()

            @pl.when(j < CPW // 2 - 1)
            def _():
                pltpu.async_copy(g_hbm.at[src_v.at[i0 + 2]], rows0, sem0)

            pltpu.sync_copy(rows1, acc.at[dst_v.at[i0 + 1]], add=True)
            return 0

        lax.fori_loop(0, CPW // 2, body, 0)
        plsc.subcore_barrier()
        pltpu.sync_copy(
            acc.at[pl.ds(s * RPT, RPT)], out_hbm.at[c, ph, pl.ds(s * RPT, RPT)]
        )
        plsc.subcore_barrier()


def _dis_block(d_ref):
    deg = d_ref[0][:, :1] + d_ref[1][:, :1]
    return lax.rsqrt(deg + 1.0)


_RB = 2000


def _tc_prep_body(x_ref, w_ref, d_ref, o_ref):
    dis = _dis_block(d_ref)
    g = jnp.dot(x_ref[...], w_ref[...], preferred_element_type=jnp.float32)
    o_ref[...] = g * dis


def _tc_prep(x, W1, d2):
    return pl.pallas_call(
        _tc_prep_body,
        grid=(N // _RB,),
        in_specs=[
            pl.BlockSpec((_RB, H), lambda i: (i, 0)),
            pl.BlockSpec((H, H), lambda i: (0, 0)),
            pl.BlockSpec((NC, _RB, H), lambda i: (0, i, 0)),
        ],
        out_specs=pl.BlockSpec((_RB, H), lambda i: (i, 0)),
        out_shape=jax.ShapeDtypeStruct((N, H), jnp.float32),
    )(x, W1, d2)


def _tc_mid_body(p_ref, g_ref, d_ref, b_ref, w_ref, o_ref):
    dis = _dis_block(d_ref)
    h = dis * (p_ref[0] + p_ref[1] + g_ref[...]) + b_ref[...]
    h = jnp.maximum(h, 0.0)
    o_ref[...] = jnp.dot(h, w_ref[...], preferred_element_type=jnp.float32) * dis


def _tc_mid(p, g, d2, b, Wn):
    return pl.pallas_call(
        _tc_mid_body,
        grid=(N // _RB,),
        in_specs=[
            pl.BlockSpec((NC, _RB, H), lambda i: (0, i, 0)),
            pl.BlockSpec((_RB, H), lambda i: (i, 0)),
            pl.BlockSpec((NC, _RB, H), lambda i: (0, i, 0)),
            pl.BlockSpec((1, H), lambda i: (0, 0)),
            pl.BlockSpec((H, H), lambda i: (0, 0)),
        ],
        out_specs=pl.BlockSpec((_RB, H), lambda i: (i, 0)),
        out_shape=jax.ShapeDtypeStruct((N, H), jnp.float32),
    )(p, g, d2, b, Wn)


def _tc_final_body(p_ref, g_ref, d_ref, b_ref, batch_ref, wl_ref, bl_ref, o_ref):
    dis = _dis_block(d_ref)
    h = dis * (p_ref[0] + p_ref[1] + g_ref[...]) + b_ref[...]
    seg = lax.broadcasted_iota(jnp.int32, (G, N), 0)
    mask = (batch_ref[...] == seg).astype(jnp.float32)
    sums = jnp.dot(mask, h, preferred_element_type=jnp.float32)
    cnt = jnp.maximum(jnp.sum(mask, axis=1, keepdims=True), 1.0)
    o_ref[...] = (
        jnp.dot(sums / cnt, wl_ref[...], preferred_element_type=jnp.float32)
        + bl_ref[...]
    )


def _tc_final(p, g, d2, b, batch2, Wl, bl):
    return pl.pallas_call(
        _tc_final_body,
        grid=(1,),
        in_specs=[
            pl.BlockSpec((NC, N, H), lambda i: (0, 0, 0)),
            pl.BlockSpec((N, H), lambda i: (0, 0)),
            pl.BlockSpec((NC, N, H), lambda i: (0, 0, 0)),
            pl.BlockSpec((1, H), lambda i: (0, 0)),
            pl.BlockSpec((1, N), lambda i: (0, 0)),
            pl.BlockSpec((H, C), lambda i: (0, 0)),
            pl.BlockSpec((1, C), lambda i: (0, 0)),
        ],
        out_specs=pl.BlockSpec((G, C), lambda i: (0, 0)),
        out_shape=jax.ShapeDtypeStruct((G, C), jnp.float32),
    )(p, g, d2, b, batch2, Wl, bl)


def kernel(x, edge_index, batch, W1, b1, W2, b2, W3, b3, Wl, bl):
    pad = EP - E
    src_p = jnp.concatenate(
        [edge_index[0], jnp.zeros((pad,), edge_index.dtype)]
    ).reshape(NW, CPW, K)
    dst_p = jnp.concatenate(
        [edge_index[1], jnp.full((pad,), N, edge_index.dtype)]
    ).reshape(NW, CPW, K)
    dst0 = jnp.where(dst_p < PH_ROWS, dst_p, TRASH)
    dst1 = jnp.where(dst_p >= PH_ROWS, dst_p - PH_ROWS, TRASH)

    ones_g = jnp.ones((8, H), jnp.float32)
    zsrc = jnp.zeros_like(src_p)
    d2 = _sc_scatter(ones_g, zsrc, dst0, dst1).reshape(NC, NPH * PH_ROWS, H)
    g1 = _tc_prep(x, W1, d2)
    p1 = _sc_scatter(g1, src_p, dst0, dst1).reshape(NC, NPH * PH_ROWS, H)
    g2 = _tc_mid(p1, g1, d2, b1.reshape(1, H), W2)
    p2 = _sc_scatter(g2, src_p, dst0, dst1).reshape(NC, NPH * PH_ROWS, H)
    g3 = _tc_mid(p2, g2, d2, b2.reshape(1, H), W3)
    p3 = _sc_scatter(g3, src_p, dst0, dst1).reshape(NC, NPH * PH_ROWS, H)
    return _tc_final(
        p3, g3, d2, b3.reshape(1, H), batch.reshape(1, N), Wl, bl.reshape(1, C)
    )


# speedup vs baseline: 9.0561x; 9.0561x over previous
import functools

import jax
import jax.numpy as jnp
from jax import lax
from jax.experimental import pallas as pl
from jax.experimental.pallas import tpu as pltpu
from jax.experimental.pallas import tpu_sc as plsc

N = 10000
E = 320000
H = 128
C = 16
G = 64

NC = 2
NS = 16
NW = NC * NS
K = 128
CPW = 80
EP = NW * CPW * K
NPH = 2
PH_ROWS = 5120
TRASH = PH_ROWS
ACC_ROWS = 5248
RPT = PH_ROWS // NS
DEG_ROWS = 10240
DEG_RPT = DEG_ROWS // NS

_mesh = plsc.VectorSubcoreMesh(
    core_axis_name="c", subcore_axis_name="s", num_cores=NC, num_subcores=NS
)


@functools.partial(
    pl.kernel,
    out_type=jax.ShapeDtypeStruct((NC, NPH, PH_ROWS, H), jnp.float32),
    mesh=_mesh,
    scratch_types=[
        pltpu.VMEM((CPW, K), jnp.int32),
        pltpu.VMEM((K, H), jnp.float32),
        pltpu.VMEM((128, H), jnp.float32),
        pltpu.VMEM_SHARED((ACC_ROWS, H), jnp.float32),
    ],
)
def _sc_deg(dst0_hbm, dst1_hbm, out_hbm, dst_v, ones_v, zb_v, acc):
    c = lax.axis_index("c")
    s = lax.axis_index("s")
    w = s * NC + c

    def fill_ones(r, _):
        for gcol in range(H // 16):
            ones_v[r, pl.ds(gcol * 16, 16)] = jnp.ones((16,), jnp.float32)
        return 0

    lax.fori_loop(0, K, fill_ones, 0)

    def fill_zero(r, _):
        for gcol in range(H // 16):
            zb_v[r, pl.ds(gcol * 16, 16)] = jnp.zeros((16,), jnp.float32)
        return 0

    lax.fori_loop(0, 128, fill_zero, 0)

    for ph, dst_hbm in enumerate((dst0_hbm, dst1_hbm)):
        pltpu.sync_copy(dst_hbm.at[w], dst_v)
        pltpu.sync_copy(zb_v, acc.at[pl.ds(s * RPT, 128)])
        pltpu.sync_copy(zb_v, acc.at[pl.ds(s * RPT + 128, 128)])
        pltpu.sync_copy(zb_v.at[pl.ds(0, 64)], acc.at[pl.ds(s * RPT + 256, 64)])
        plsc.subcore_barrier()

        def body(i, _):
            pltpu.sync_copy(ones_v, acc.at[dst_v.at[i]], add=True)
            return 0

        lax.fori_loop(0, CPW, body, 0)
        plsc.subcore_barrier()
        pltpu.sync_copy(
            acc.at[pl.ds(s * RPT, RPT)], out_hbm.at[c, ph, pl.ds(s * RPT, RPT)]
        )
        plsc.subcore_barrier()


@functools.partial(
    pl.kernel,
    out_type=jax.ShapeDtypeStruct((NC, NPH, PH_ROWS, H), jnp.float32),
    mesh=_mesh,
    scratch_types=[
        pltpu.VMEM((CPW, K), jnp.int32),
        pltpu.VMEM((CPW, K), jnp.int32),
        pltpu.VMEM((K, H), jnp.float32),
        pltpu.VMEM((K, H), jnp.float32),
        pltpu.VMEM((128, H), jnp.float32),
        pltpu.VMEM_SHARED((ACC_ROWS, H), jnp.float32),
        pltpu.SemaphoreType.DMA,
        pltpu.SemaphoreType.DMA,
    ],
)
def _sc_scatter(g_hbm, src_hbm, dst0_hbm, dst1_hbm, out_hbm,
                src_v, dst_v, rows0, rows1, zb_v, acc, sem0, sem1):
    c = lax.axis_index("c")
    s = lax.axis_index("s")
    w = s * NC + c

    def fill_zero(r, _):
        for gcol in range(H // 16):
            zb_v[r, pl.ds(gcol * 16, 16)] = jnp.zeros((16,), jnp.float32)
        return 0

    lax.fori_loop(0, 128, fill_zero, 0)

    pltpu.sync_copy(src_hbm.at[w], src_v)

    for ph, dst_hbm in enumerate((dst0_hbm, dst1_hbm)):
        pltpu.sync_copy(dst_hbm.at[w], dst_v)
        pltpu.sync_copy(zb_v, acc.at[pl.ds(s * RPT, 128)])
        pltpu.sync_copy(zb_v, acc.at[pl.ds(s * RPT + 128, 128)])
        pltpu.sync_copy(zb_v.at[pl.ds(0, 64)], acc.at[pl.ds(s * RPT + 256, 64)])
        plsc.subcore_barrier()

        pltpu.async_copy(g_hbm.at[src_v.at[0]], rows0, sem0)

        def body(j, _):
            i0 = 2 * j
            pltpu.make_async_copy(g_hbm.at[src_v.at[i0]], rows0, sem0).wait()
            pltpu.async_copy(g_hbm.at[src_v.at[i0 + 1]], rows1, sem1)
            pltpu.sync_copy(rows0, acc.at[dst_v.at[i0]], add=True)
            pltpu.make_async_copy(g_hbm.at[src_v.at[i0 + 1]], rows1, sem1).wait()

            @pl.when(j < CPW // 2 - 1)
            def _():
                pltpu.async_copy(g_hbm.at[src_v.at[i0 + 2]], rows0, sem0)

            pltpu.sync_copy(rows1, acc.at[dst_v.at[i0 + 1]], add=True)
            return 0

        lax.fori_loop(0, CPW // 2, body, 0)
        plsc.subcore_barrier()
        pltpu.sync_copy(
            acc.at[pl.ds(s * RPT, RPT)], out_hbm.at[c, ph, pl.ds(s * RPT, RPT)]
        )
        plsc.subcore_barrier()


def _dis_block(d_ref):
    deg = d_ref[0][:, :1] + d_ref[1][:, :1]
    return lax.rsqrt(deg + 1.0)


_RB = 2000


def _tc_prep_body(x_ref, w_ref, d_ref, o_ref):
    dis = _dis_block(d_ref)
    g = jnp.dot(x_ref[...], w_ref[...], preferred_element_type=jnp.float32)
    o_ref[...] = g * dis


def _tc_prep(x, W1, d2):
    return pl.pallas_call(
        _tc_prep_body,
        grid=(N // _RB,),
        in_specs=[
            pl.BlockSpec((_RB, H), lambda i: (i, 0)),
            pl.BlockSpec((H, H), lambda i: (0, 0)),
            pl.BlockSpec((NC, _RB, H), lambda i: (0, i, 0)),
        ],
        out_specs=pl.BlockSpec((_RB, H), lambda i: (i, 0)),
        out_shape=jax.ShapeDtypeStruct((N, H), jnp.float32),
    )(x, W1, d2)


def _tc_mid_body(p_ref, g_ref, d_ref, b_ref, w_ref, o_ref):
    dis = _dis_block(d_ref)
    h = dis * (p_ref[0] + p_ref[1] + g_ref[...]) + b_ref[...]
    h = jnp.maximum(h, 0.0)
    o_ref[...] = jnp.dot(h, w_ref[...], preferred_element_type=jnp.float32) * dis


def _tc_mid(p, g, d2, b, Wn):
    return pl.pallas_call(
        _tc_mid_body,
        grid=(N // _RB,),
        in_specs=[
            pl.BlockSpec((NC, _RB, H), lambda i: (0, i, 0)),
            pl.BlockSpec((_RB, H), lambda i: (i, 0)),
            pl.BlockSpec((NC, _RB, H), lambda i: (0, i, 0)),
            pl.BlockSpec((1, H), lambda i: (0, 0)),
            pl.BlockSpec((H, H), lambda i: (0, 0)),
        ],
        out_specs=pl.BlockSpec((_RB, H), lambda i: (i, 0)),
        out_shape=jax.ShapeDtypeStruct((N, H), jnp.float32),
    )(p, g, d2, b, Wn)


def _tc_final_body(p_ref, g_ref, d_ref, b_ref, batch_ref, wl_ref, bl_ref, o_ref):
    dis = _dis_block(d_ref)
    h = dis * (p_ref[0] + p_ref[1] + g_ref[...]) + b_ref[...]
    seg = lax.broadcasted_iota(jnp.int32, (G, N), 0)
    mask = (batch_ref[...] == seg).astype(jnp.float32)
    sums = jnp.dot(mask, h, preferred_element_type=jnp.float32)
    cnt = jnp.maximum(jnp.sum(mask, axis=1, keepdims=True), 1.0)
    o_ref[...] = (
        jnp.dot(sums / cnt, wl_ref[...], preferred_element_type=jnp.float32)
        + bl_ref[...]
    )


def _tc_final(p, g, d2, b, batch2, Wl, bl):
    return pl.pallas_call(
        _tc_final_body,
        grid=(1,),
        in_specs=[
            pl.BlockSpec((NC, N, H), lambda i: (0, 0, 0)),
            pl.BlockSpec((N, H), lambda i: (0, 0)),
            pl.BlockSpec((NC, N, H), lambda i: (0, 0, 0)),
            pl.BlockSpec((1, H), lambda i: (0, 0)),
            pl.BlockSpec((1, N), lambda i: (0, 0)),
            pl.BlockSpec((H, C), lambda i: (0, 0)),
            pl.BlockSpec((1, C), lambda i: (0, 0)),
        ],
        out_specs=pl.BlockSpec((G, C), lambda i: (0, 0)),
        out_shape=jax.ShapeDtypeStruct((G, C), jnp.float32),
    )(p, g, d2, b, batch2, Wl, bl)


def kernel(x, edge_index, batch, W1, b1, W2, b2, W3, b3, Wl, bl):
    pad = EP - E
    src_p = jnp.concatenate(
        [edge_index[0], jnp.zeros((pad,), edge_index.dtype)]
    ).reshape(NW, CPW, K)
    dst_p = jnp.concatenate(
        [edge_index[1], jnp.full((pad,), N, edge_index.dtype)]
    ).reshape(NW, CPW, K)
    dst0 = jnp.where(dst_p < PH_ROWS, dst_p, TRASH)
    dst1 = jnp.where(dst_p >= PH_ROWS, dst_p - PH_ROWS, TRASH)

    d2 = _sc_deg(dst0, dst1).reshape(NC, NPH * PH_ROWS, H)
    g1 = _tc_prep(x, W1, d2)
    p1 = _sc_scatter(g1, src_p, dst0, dst1).reshape(NC, NPH * PH_ROWS, H)
    g2 = _tc_mid(p1, g1, d2, b1.reshape(1, H), W2)
    p2 = _sc_scatter(g2, src_p, dst0, dst1).reshape(NC, NPH * PH_ROWS, H)
    g3 = _tc_mid(p2, g2, d2, b2.reshape(1, H), W3)
    p3 = _sc_scatter(g3, src_p, dst0, dst1).reshape(NC, NPH * PH_ROWS, H)
    return _tc_final(
        p3, g3, d2, b3.reshape(1, H), batch.reshape(1, N), Wl, bl.reshape(1, C)
    )


# speedup vs baseline: 9.3679x; 1.0344x over previous
import functools

import jax
import jax.numpy as jnp
from jax import lax
from jax.experimental import pallas as pl
from jax.experimental.pallas import tpu as pltpu
from jax.experimental.pallas import tpu_sc as plsc

N = 10000
E = 320000
H = 128
C = 16
G = 64

NC = 2
NS = 16
NW = NC * NS
K = 128
CPW = 80
EP = NW * CPW * K
NPH = 2
PH_ROWS = 5120
TRASH = PH_ROWS
ACC_ROWS = 5248
RPT = PH_ROWS // NS
DEG_ROWS = 10240
DEG_RPT = DEG_ROWS // NS

_mesh = plsc.VectorSubcoreMesh(
    core_axis_name="c", subcore_axis_name="s", num_cores=NC, num_subcores=NS
)


@functools.partial(
    pl.kernel,
    out_type=jax.ShapeDtypeStruct((NC, NPH, PH_ROWS, H), jnp.float32),
    mesh=_mesh,
    scratch_types=[
        pltpu.VMEM((CPW, K), jnp.int32),
        pltpu.VMEM((K, H), jnp.float32),
        pltpu.VMEM((128, H), jnp.float32),
        pltpu.VMEM_SHARED((ACC_ROWS, H), jnp.float32),
    ],
)
def _sc_deg(dst0_hbm, dst1_hbm, out_hbm, dst_v, ones_v, zb_v, acc):
    c = lax.axis_index("c")
    s = lax.axis_index("s")
    w = s * NC + c

    def fill_ones(r, _):
        for gcol in range(H // 16):
            ones_v[r, pl.ds(gcol * 16, 16)] = jnp.ones((16,), jnp.float32)
        return 0

    lax.fori_loop(0, K, fill_ones, 0)

    def fill_zero(r, _):
        for gcol in range(H // 16):
            zb_v[r, pl.ds(gcol * 16, 16)] = jnp.zeros((16,), jnp.float32)
        return 0

    lax.fori_loop(0, 128, fill_zero, 0)

    for ph, dst_hbm in enumerate((dst0_hbm, dst1_hbm)):
        pltpu.sync_copy(dst_hbm.at[w], dst_v)
        pltpu.sync_copy(zb_v, acc.at[pl.ds(s * RPT, 128)])
        pltpu.sync_copy(zb_v, acc.at[pl.ds(s * RPT + 128, 128)])
        pltpu.sync_copy(zb_v.at[pl.ds(0, 64)], acc.at[pl.ds(s * RPT + 256, 64)])
        plsc.subcore_barrier()

        def body(i, _):
            pltpu.sync_copy(ones_v, acc.at[dst_v.at[i]], add=True)
            return 0

        lax.fori_loop(0, CPW, body, 0)
        plsc.subcore_barrier()
        pltpu.sync_copy(
            acc.at[pl.ds(s * RPT, RPT)], out_hbm.at[c, ph, pl.ds(s * RPT, RPT)]
        )
        plsc.subcore_barrier()


@functools.partial(
    pl.kernel,
    out_type=jax.ShapeDtypeStruct((NC, NPH, PH_ROWS, H), jnp.float32),
    mesh=_mesh,
    scratch_types=[
        pltpu.VMEM((CPW, K), jnp.int32),
        pltpu.VMEM((CPW, K), jnp.int32),
        pltpu.VMEM((K, H), jnp.float32),
        pltpu.VMEM((K, H), jnp.float32),
        pltpu.VMEM((128, H), jnp.float32),
        pltpu.VMEM_SHARED((ACC_ROWS, H), jnp.float32),
        pltpu.SemaphoreType.DMA,
        pltpu.SemaphoreType.DMA,
    ],
)
def _sc_scatter(g_hbm, src_hbm, dst0_hbm, dst1_hbm, out_hbm,
                src_v, dst_v, rows0, rows1, zb_v, acc, sem0, sem1):
    c = lax.axis_index("c")
    s = lax.axis_index("s")
    w = s * NC + c

    def fill_zero(r, _):
        for gcol in range(H // 16):
            zb_v[r, pl.ds(gcol * 16, 16)] = jnp.zeros((16,), jnp.float32)
        return 0

    lax.fori_loop(0, 128, fill_zero, 0)

    pltpu.sync_copy(src_hbm.at[w], src_v)

    for ph, dst_hbm in enumerate((dst0_hbm, dst1_hbm)):
        pltpu.sync_copy(dst_hbm.at[w], dst_v)
        pltpu.sync_copy(zb_v, acc.at[pl.ds(s * RPT, 128)])
        pltpu.sync_copy(zb_v, acc.at[pl.ds(s * RPT + 128, 128)])
        pltpu.sync_copy(zb_v.at[pl.ds(0, 64)], acc.at[pl.ds(s * RPT + 256, 64)])
        plsc.subcore_barrier()

        pltpu.async_copy(g_hbm.at[src_v.at[0]], rows0, sem0)

        def body(j, _):
            i0 = 2 * j
            pltpu.make_async_copy(g_hbm.at[src_v.at[i0]], rows0, sem0).wait()
            pltpu.async_copy(g_hbm.at[src_v.at[i0 + 1]], rows1, sem1)
            pltpu.sync_copy(rows0, acc.at[dst_v.at[i0]], add=True)
            pltpu.make_async_copy(g_hbm.at[src_v.at[i0 + 1]], rows1, sem1).wait()

            @pl.when(j < CPW // 2 - 1)
            def _():
                pltpu.async_copy(g_hbm.at[src_v.at[i0 + 2]], rows0, sem0)

            pltpu.sync_copy(rows1, acc.at[dst_v.at[i0 + 1]], add=True)
            return 0

        lax.fori_loop(0, CPW // 2, body, 0)
        plsc.subcore_barrier()
        pltpu.sync_copy(
            acc.at[pl.ds(s * RPT, RPT)], out_hbm.at[c, ph, pl.ds(s * RPT, RPT)]
        )
        plsc.subcore_barrier()


def _dis_block(d_ref):
    deg = d_ref[0][:, :1] + d_ref[1][:, :1]
    return lax.rsqrt(deg + 1.0)


_RB = 2000


def _tc_prep_body(x_ref, w_ref, d_ref, o_ref):
    dis = _dis_block(d_ref)
    g = jnp.dot(x_ref[...], w_ref[...], preferred_element_type=jnp.float32)
    o_ref[...] = g * dis


def _tc_prep(x, W1, d2):
    return pl.pallas_call(
        _tc_prep_body,
        grid=(N // _RB,),
        in_specs=[
            pl.BlockSpec((_RB, H), lambda i: (i, 0)),
            pl.BlockSpec((H, H), lambda i: (0, 0)),
            pl.BlockSpec((NC, _RB, H), lambda i: (0, i, 0)),
        ],
        out_specs=pl.BlockSpec((_RB, H), lambda i: (i, 0)),
        out_shape=jax.ShapeDtypeStruct((N, H), jnp.float32),
    )(x, W1, d2)


def _tc_mid_body(p_ref, g_ref, d_ref, b_ref, w_ref, o_ref):
    dis = _dis_block(d_ref)
    h = dis * (p_ref[0] + p_ref[1] + g_ref[...]) + b_ref[...]
    h = jnp.maximum(h, 0.0)
    o_ref[...] = jnp.dot(h, w_ref[...], preferred_element_type=jnp.float32) * dis


def _tc_mid(p, g, d2, b, Wn):
    return pl.pallas_call(
        _tc_mid_body,
        grid=(N // _RB,),
        in_specs=[
            pl.BlockSpec((NC, _RB, H), lambda i: (0, i, 0)),
            pl.BlockSpec((_RB, H), lambda i: (i, 0)),
            pl.BlockSpec((NC, _RB, H), lambda i: (0, i, 0)),
            pl.BlockSpec((1, H), lambda i: (0, 0)),
            pl.BlockSpec((H, H), lambda i: (0, 0)),
        ],
        out_specs=pl.BlockSpec((_RB, H), lambda i: (i, 0)),
        out_shape=jax.ShapeDtypeStruct((N, H), jnp.float32),
    )(p, g, d2, b, Wn)


def _tc_final_body(p_ref, g_ref, d_ref, b_ref, batch_ref, wl_ref, bl_ref, o_ref):
    dis = _dis_block(d_ref)
    h = dis * (p_ref[0] + p_ref[1] + g_ref[...]) + b_ref[...]
    seg = lax.broadcasted_iota(jnp.int32, (G, N), 0)
    mask = (batch_ref[...] == seg).astype(jnp.float32)
    sums = jnp.dot(mask, h, preferred_element_type=jnp.float32)
    cnt = jnp.maximum(jnp.sum(mask, axis=1, keepdims=True), 1.0)
    o_ref[...] = (
        jnp.dot(sums / cnt, wl_ref[...], preferred_element_type=jnp.float32)
        + bl_ref[...]
    )


def _tc_final(p, g, d2, b, batch2, Wl, bl):
    return pl.pallas_call(
        _tc_final_body,
        grid=(1,),
        in_specs=[
            pl.BlockSpec((NC, N, H), lambda i: (0, 0, 0)),
            pl.BlockSpec((N, H), lambda i: (0, 0)),
            pl.BlockSpec((NC, N, H), lambda i: (0, 0, 0)),
            pl.BlockSpec((1, H), lambda i: (0, 0)),
            pl.BlockSpec((1, N), lambda i: (0, 0)),
            pl.BlockSpec((H, C), lambda i: (0, 0)),
            pl.BlockSpec((1, C), lambda i: (0, 0)),
        ],
        out_specs=pl.BlockSpec((G, C), lambda i: (0, 0)),
        out_shape=jax.ShapeDtypeStruct((G, C), jnp.float32),
    )(p, g, d2, b, batch2, Wl, bl)


def kernel(x, edge_index, batch, W1, b1, W2, b2, W3, b3, Wl, bl):
    pad = EP - E
    src_p = jnp.concatenate(
        [edge_index[0], jnp.zeros((pad,), edge_index.dtype)]
    ).reshape(NW, CPW, K)
    dst_p = jnp.concatenate(
        [edge_index[1], jnp.full((pad,), N, edge_index.dtype)]
    ).reshape(NW, CPW, K)
    trash = TRASH + (dst_p & 127)
    dst0 = jnp.where(dst_p < PH_ROWS, dst_p, trash)
    dst1 = jnp.where(dst_p >= PH_ROWS, dst_p - PH_ROWS, trash)

    d2 = _sc_deg(dst0, dst1).reshape(NC, NPH * PH_ROWS, H)
    g1 = _tc_prep(x, W1, d2)
    p1 = _sc_scatter(g1, src_p, dst0, dst1).reshape(NC, NPH * PH_ROWS, H)
    g2 = _tc_mid(p1, g1, d2, b1.reshape(1, H), W2)
    p2 = _sc_scatter(g2, src_p, dst0, dst1).reshape(NC, NPH * PH_ROWS, H)
    g3 = _tc_mid(p2, g2, d2, b2.reshape(1, H), W3)
    p3 = _sc_scatter(g3, src_p, dst0, dst1).reshape(NC, NPH * PH_ROWS, H)
    return _tc_final(
        p3, g3, d2, b3.reshape(1, H), batch.reshape(1, N), Wl, bl.reshape(1, C)
    )


# speedup vs baseline: 12.6631x; 1.3518x over previous
import functools

import jax
import jax.numpy as jnp
from jax import lax
from jax.experimental import pallas as pl
from jax.experimental.pallas import tpu as pltpu
from jax.experimental.pallas import tpu_sc as plsc

N = 10000
E = 320000
H = 128
C = 16
G = 64

NC = 2
NS = 16
NW = NC * NS
K = 128
KC0 = 36
KC1 = 122
KCM = 122
NPH = 2
PH_ROWS = 5120
TRASH = PH_ROWS
ACC_ROWS = 5248
RPT = PH_ROWS // NS

_mesh = plsc.VectorSubcoreMesh(
    core_axis_name="c", subcore_axis_name="s", num_cores=NC, num_subcores=NS
)


@functools.partial(
    pl.kernel,
    out_type=jax.ShapeDtypeStruct((NC, NPH, PH_ROWS, H), jnp.float32),
    mesh=_mesh,
    scratch_types=[
        pltpu.VMEM((KCM, K), jnp.int32),
        pltpu.VMEM((K, H), jnp.float32),
        pltpu.VMEM((128, H), jnp.float32),
        pltpu.VMEM_SHARED((ACC_ROWS, H), jnp.float32),
    ],
)
def _sc_deg(dst0_hbm, dst1_hbm, out_hbm, dst_v, ones_v, zb_v, acc):
    c = lax.axis_index("c")
    s = lax.axis_index("s")
    w = s * NC + c
    ncw = jnp.where(c == 0, KC0, KC1)

    def fill_ones(r, _):
        for gcol in range(H // 16):
            ones_v[r, pl.ds(gcol * 16, 16)] = jnp.ones((16,), jnp.float32)
        return 0

    lax.fori_loop(0, K, fill_ones, 0)

    def fill_zero(r, _):
        for gcol in range(H // 16):
            zb_v[r, pl.ds(gcol * 16, 16)] = jnp.zeros((16,), jnp.float32)
        return 0

    lax.fori_loop(0, 128, fill_zero, 0)

    for ph, dst_hbm in enumerate((dst0_hbm, dst1_hbm)):
        pltpu.sync_copy(dst_hbm.at[w], dst_v)
        pltpu.sync_copy(zb_v, acc.at[pl.ds(s * RPT, 128)])
        pltpu.sync_copy(zb_v, acc.at[pl.ds(s * RPT + 128, 128)])
        pltpu.sync_copy(zb_v.at[pl.ds(0, 64)], acc.at[pl.ds(s * RPT + 256, 64)])
        plsc.subcore_barrier()

        def body(i, _):
            pltpu.sync_copy(ones_v, acc.at[dst_v.at[i]], add=True)
            return 0

        lax.fori_loop(0, ncw, body, 0)
        plsc.subcore_barrier()
        pltpu.sync_copy(
            acc.at[pl.ds(s * RPT, RPT)], out_hbm.at[c, ph, pl.ds(s * RPT, RPT)]
        )
        plsc.subcore_barrier()


@functools.partial(
    pl.kernel,
    out_type=jax.ShapeDtypeStruct((NC, NPH, PH_ROWS, H), jnp.float32),
    mesh=_mesh,
    scratch_types=[
        pltpu.VMEM((KCM, K), jnp.int32),
        pltpu.VMEM((KCM, K), jnp.int32),
        pltpu.VMEM((K, H), jnp.float32),
        pltpu.VMEM((K, H), jnp.float32),
        pltpu.VMEM((128, H), jnp.float32),
        pltpu.VMEM_SHARED((ACC_ROWS, H), jnp.float32),
        pltpu.SemaphoreType.DMA,
        pltpu.SemaphoreType.DMA,
    ],
)
def _sc_scatter(g_hbm, src_hbm, dst0_hbm, dst1_hbm, out_hbm,
                src_v, dst_v, rows0, rows1, zb_v, acc, sem0, sem1):
    c = lax.axis_index("c")
    s = lax.axis_index("s")
    w = s * NC + c
    npair = jnp.where(c == 0, KC0 // 2, KC1 // 2)

    def fill_zero(r, _):
        for gcol in range(H // 16):
            zb_v[r, pl.ds(gcol * 16, 16)] = jnp.zeros((16,), jnp.float32)
        return 0

    lax.fori_loop(0, 128, fill_zero, 0)

    pltpu.sync_copy(src_hbm.at[w], src_v)

    for ph, dst_hbm in enumerate((dst0_hbm, dst1_hbm)):
        pltpu.sync_copy(dst_hbm.at[w], dst_v)
        pltpu.sync_copy(zb_v, acc.at[pl.ds(s * RPT, 128)])
        pltpu.sync_copy(zb_v, acc.at[pl.ds(s * RPT + 128, 128)])
        pltpu.sync_copy(zb_v.at[pl.ds(0, 64)], acc.at[pl.ds(s * RPT + 256, 64)])
        plsc.subcore_barrier()

        pltpu.async_copy(g_hbm.at[src_v.at[0]], rows0, sem0)

        def body(j, _):
            i0 = 2 * j
            pltpu.make_async_copy(g_hbm.at[src_v.at[i0]], rows0, sem0).wait()
            pltpu.async_copy(g_hbm.at[src_v.at[i0 + 1]], rows1, sem1)
            pltpu.sync_copy(rows0, acc.at[dst_v.at[i0]], add=True)
            pltpu.make_async_copy(g_hbm.at[src_v.at[i0 + 1]], rows1, sem1).wait()

            @pl.when(j < npair - 1)
            def _():
                pltpu.async_copy(g_hbm.at[src_v.at[i0 + 2]], rows0, sem0)

            pltpu.sync_copy(rows1, acc.at[dst_v.at[i0 + 1]], add=True)
            return 0

        lax.fori_loop(0, npair, body, 0)
        plsc.subcore_barrier()
        pltpu.sync_copy(
            acc.at[pl.ds(s * RPT, RPT)], out_hbm.at[c, ph, pl.ds(s * RPT, RPT)]
        )
        plsc.subcore_barrier()


def _dis_block(d_ref):
    deg = d_ref[0][:, :1] + d_ref[1][:, :1]
    return lax.rsqrt(deg + 1.0)


_RB = 2000


def _tc_prep_body(x_ref, w_ref, d_ref, o_ref):
    dis = _dis_block(d_ref)
    g = jnp.dot(x_ref[...], w_ref[...], preferred_element_type=jnp.float32)
    o_ref[...] = g * dis


def _tc_prep(x, W1, d2):
    return pl.pallas_call(
        _tc_prep_body,
        grid=(N // _RB,),
        in_specs=[
            pl.BlockSpec((_RB, H), lambda i: (i, 0)),
            pl.BlockSpec((H, H), lambda i: (0, 0)),
            pl.BlockSpec((NC, _RB, H), lambda i: (0, i, 0)),
        ],
        out_specs=pl.BlockSpec((_RB, H), lambda i: (i, 0)),
        out_shape=jax.ShapeDtypeStruct((N, H), jnp.float32),
    )(x, W1, d2)


def _tc_mid_body(p_ref, g_ref, d_ref, b_ref, w_ref, o_ref):
    dis = _dis_block(d_ref)
    h = dis * (p_ref[0] + p_ref[1] + g_ref[...]) + b_ref[...]
    h = jnp.maximum(h, 0.0)
    o_ref[...] = jnp.dot(h, w_ref[...], preferred_element_type=jnp.float32) * dis


def _tc_mid(p, g, d2, b, Wn):
    return pl.pallas_call(
        _tc_mid_body,
        grid=(N // _RB,),
        in_specs=[
            pl.BlockSpec((NC, _RB, H), lambda i: (0, i, 0)),
            pl.BlockSpec((_RB, H), lambda i: (i, 0)),
            pl.BlockSpec((NC, _RB, H), lambda i: (0, i, 0)),
            pl.BlockSpec((1, H), lambda i: (0, 0)),
            pl.BlockSpec((H, H), lambda i: (0, 0)),
        ],
        out_specs=pl.BlockSpec((_RB, H), lambda i: (i, 0)),
        out_shape=jax.ShapeDtypeStruct((N, H), jnp.float32),
    )(p, g, d2, b, Wn)


def _tc_final_body(p_ref, g_ref, d_ref, b_ref, batch_ref, wl_ref, bl_ref, o_ref):
    dis = _dis_block(d_ref)
    h = dis * (p_ref[0] + p_ref[1] + g_ref[...]) + b_ref[...]
    seg = lax.broadcasted_iota(jnp.int32, (G, N), 0)
    mask = (batch_ref[...] == seg).astype(jnp.float32)
    sums = jnp.dot(mask, h, preferred_element_type=jnp.float32)
    cnt = jnp.maximum(jnp.sum(mask, axis=1, keepdims=True), 1.0)
    o_ref[...] = (
        jnp.dot(sums / cnt, wl_ref[...], preferred_element_type=jnp.float32)
        + bl_ref[...]
    )


def _tc_final(p, g, d2, b, batch2, Wl, bl):
    return pl.pallas_call(
        _tc_final_body,
        grid=(1,),
        in_specs=[
            pl.BlockSpec((NC, N, H), lambda i: (0, 0, 0)),
            pl.BlockSpec((N, H), lambda i: (0, 0)),
            pl.BlockSpec((NC, N, H), lambda i: (0, 0, 0)),
            pl.BlockSpec((1, H), lambda i: (0, 0)),
            pl.BlockSpec((1, N), lambda i: (0, 0)),
            pl.BlockSpec((H, C), lambda i: (0, 0)),
            pl.BlockSpec((1, C), lambda i: (0, 0)),
        ],
        out_specs=pl.BlockSpec((G, C), lambda i: (0, 0)),
        out_shape=jax.ShapeDtypeStruct((G, C), jnp.float32),
    )(p, g, d2, b, batch2, Wl, bl)


def _distribute(v, padval):
    e0 = 16 * KC0 * K
    e1cap = 16 * KC1 * K
    p0 = jnp.concatenate(
        [
            v[:e0].reshape(16, KC0 * K),
            jnp.full((16, (KCM - KC0) * K), padval, v.dtype),
        ],
        axis=1,
    )
    p1 = jnp.concatenate(
        [v[e0:], jnp.full((e1cap - (E - e0),), padval, v.dtype)]
    ).reshape(16, KC1 * K)
    p1 = jnp.concatenate(
        [p1, jnp.full((16, (KCM - KC1) * K), padval, v.dtype)], axis=1
    )
    return jnp.stack([p0, p1], axis=1).reshape(NW, KCM, K)


def kernel(x, edge_index, batch, W1, b1, W2, b2, W3, b3, Wl, bl):
    src_p = _distribute(edge_index[0], jnp.int32(0))
    dst_p = _distribute(edge_index[1], jnp.int32(N))
    trash = TRASH + (dst_p & 127)
    dst0 = jnp.where(dst_p < PH_ROWS, dst_p, trash)
    dst1 = jnp.where(
        jnp.logical_and(dst_p >= PH_ROWS, dst_p < N), dst_p - PH_ROWS, trash
    )

    d2 = _sc_deg(dst0, dst1).reshape(NC, NPH * PH_ROWS, H)
    g1 = _tc_prep(x, W1, d2)
    p1 = _sc_scatter(g1, src_p, dst0, dst1).reshape(NC, NPH * PH_ROWS, H)
    g2 = _tc_mid(p1, g1, d2, b1.reshape(1, H), W2)
    p2 = _sc_scatter(g2, src_p, dst0, dst1).reshape(NC, NPH * PH_ROWS, H)
    g3 = _tc_mid(p2, g2, d2, b2.reshape(1, H), W3)
    p3 = _sc_scatter(g3, src_p, dst0, dst1).reshape(NC, NPH * PH_ROWS, H)
    return _tc_final(
        p3, g3, d2, b3.reshape(1, H), batch.reshape(1, N), Wl, bl.reshape(1, C)
    )
